# Initial kernel scaffold; baseline (speedup 1.0000x reference)
import jax
import jax.numpy as jnp
from jax.experimental import pallas as pl


def kernel(x, table):
    raise NotImplementedError("write your pallas kernel here")


# speedup vs baseline: 9.5500x; 9.5500x over previous
import functools

import jax
import jax.numpy as jnp
from jax import lax
from jax.experimental import pallas as pl
from jax.experimental.pallas import tpu as pltpu
from jax.experimental.pallas import tpu_sc as plsc

_NUM_WORKERS = 32
_G = 128
_K = 8


def _emb_kernel(idx_hbm, table_hbm, out_hbm, idx_v, rows_v, gsem,
                *, steps, rows_per_worker, d):
    wid = lax.axis_index("s") * 2 + lax.axis_index("c")
    row_base = wid * rows_per_worker
    blk = _K * _G

    @pl.loop(0, steps)
    def step(i):
        r0 = row_base + i * _K
        pltpu.sync_copy(idx_hbm.at[pl.ds(r0, _K)], idx_v)
        cps = [
            pltpu.async_copy(
                table_hbm.at[idx_v.at[j]],
                rows_v.at[pl.ds(j * _G, _G)],
                gsem,
            )
            for j in range(_K)
        ]
        for cp in cps:
            cp.wait()
        pltpu.sync_copy(rows_v, out_hbm.at[pl.ds(r0 * _G, blk)])


def kernel(x, table):
    b, l = x.shape
    v, d = table.shape
    n = b * l
    blk = _K * _G
    assert n % (_NUM_WORKERS * blk) == 0
    steps = n // (_NUM_WORKERS * blk)
    rows_per_worker = n // (_NUM_WORKERS * _G)

    idx2d = x.reshape(n // _G, _G).astype(jnp.int32)

    mesh = plsc.VectorSubcoreMesh(core_axis_name="c", subcore_axis_name="s")
    emb = pl.kernel(
        functools.partial(
            _emb_kernel, steps=steps, rows_per_worker=rows_per_worker, d=d
        ),
        out_type=jax.ShapeDtypeStruct((n, d), jnp.float32),
        mesh=mesh,
        scratch_types=[
            pltpu.VMEM((_K, _G), jnp.int32),
            pltpu.VMEM((blk, d), jnp.float32),
            pltpu.SemaphoreType.DMA,
        ],
        compiler_params=pltpu.CompilerParams(use_tc_tiling_on_sc=False),
    )
    out = emb(idx2d, table)
    return out.reshape(b, l * d)


# speedup vs baseline: 10.4494x; 1.0942x over previous
import functools

import jax
import jax.numpy as jnp
from jax import lax
from jax.experimental import pallas as pl
from jax.experimental.pallas import tpu as pltpu
from jax.experimental.pallas import tpu_sc as plsc

_NUM_WORKERS = 32
_G = 128
_K = 8


def _emb_kernel(idx_hbm, table_hbm, out_hbm, idx_v, rows_v, gsem, isem,
                ssem0, ssem1, *, steps, rows_per_worker):
    wid = lax.axis_index("s") * 2 + lax.axis_index("c")
    row_base = wid * rows_per_worker
    blk = _K * _G
    ssems = (ssem0, ssem1)

    pltpu.sync_copy(idx_hbm.at[pl.ds(row_base, _K)], idx_v.at[0])

    @pl.loop(0, steps, step=2)
    def pair(i0):
        for b in range(2):
            i = i0 + b
            r0 = row_base + i * _K
            nb = 1 - b

            @pl.when(i + 1 < steps)
            def _():
                pltpu.async_copy(
                    idx_hbm.at[pl.ds(r0 + _K, _K)], idx_v.at[nb], isem
                )

            @pl.when(i >= 2)
            def _():
                pltpu.make_async_copy(
                    rows_v.at[b], out_hbm.at[pl.ds(0, blk)], ssems[b]
                ).wait()

            @pl.when(i >= 1)
            def _():
                pltpu.make_async_copy(
                    idx_hbm.at[pl.ds(0, _K)], idx_v.at[b], isem
                ).wait()

            cps = [
                pltpu.async_copy(
                    table_hbm.at[idx_v.at[b].at[j]],
                    rows_v.at[b].at[pl.ds(j * _G, _G)],
                    gsem,
                )
                for j in range(_K)
            ]
            for cp in cps:
                cp.wait()

            pltpu.async_copy(
                rows_v.at[b], out_hbm.at[pl.ds(r0 * _G, blk)], ssems[b]
            )

    for b in range(2):
        pltpu.make_async_copy(
            rows_v.at[b], out_hbm.at[pl.ds(0, blk)], ssems[b]
        ).wait()


def kernel(x, table):
    b, l = x.shape
    v, d = table.shape
    n = b * l
    blk = _K * _G
    assert n % (_NUM_WORKERS * blk) == 0
    steps = n // (_NUM_WORKERS * blk)
    assert steps % 2 == 0
    rows_per_worker = n // (_NUM_WORKERS * _G)

    idx2d = x.reshape(n // _G, _G).astype(jnp.int32)

    mesh = plsc.VectorSubcoreMesh(core_axis_name="c", subcore_axis_name="s")
    emb = pl.kernel(
        functools.partial(
            _emb_kernel, steps=steps, rows_per_worker=rows_per_worker
        ),
        out_type=jax.ShapeDtypeStruct((n, d), jnp.float32),
        mesh=mesh,
        scratch_types=[
            pltpu.VMEM((2, _K, _G), jnp.int32),
            pltpu.VMEM((2, blk, d), jnp.float32),
            pltpu.SemaphoreType.DMA,
            pltpu.SemaphoreType.DMA,
            pltpu.SemaphoreType.DMA,
            pltpu.SemaphoreType.DMA,
        ],
        compiler_params=pltpu.CompilerParams(use_tc_tiling_on_sc=False),
    )
    out = emb(idx2d, table)
    return out.reshape(b, l * d)


# speedup vs baseline: 10.5211x; 1.0069x over previous
import functools

import jax
import jax.numpy as jnp
from jax import lax
from jax.experimental import pallas as pl
from jax.experimental.pallas import tpu as pltpu
from jax.experimental.pallas import tpu_sc as plsc

_NUM_WORKERS = 32
_G = 128
_K = 10


def _emb_kernel(idx_hbm, table_hbm, out_hbm, idx_v, rows_v, gsem0, gsem1,
                isem, ssem0, ssem1, *, steps, rows_per_worker):
    wid = lax.axis_index("s") * 2 + lax.axis_index("c")
    row_base = wid * rows_per_worker
    blk = _K * _G
    gsems = (gsem0, gsem1)
    ssems = (ssem0, ssem1)

    def issue_gathers(b):
        for j in range(_K):
            pltpu.async_copy(
                table_hbm.at[idx_v.at[b].at[j]],
                rows_v.at[b].at[pl.ds(j * _G, _G)],
                gsems[b],
            )

    def drain_gathers(b):
        for j in range(_K):
            pltpu.make_async_copy(
                table_hbm.at[idx_v.at[b].at[j]],
                rows_v.at[b].at[pl.ds(j * _G, _G)],
                gsems[b],
            ).wait()

    pltpu.sync_copy(idx_hbm.at[pl.ds(row_base, _K)], idx_v.at[0])

    @pl.loop(0, steps, step=2)
    def pair(i0):
        for b in range(2):
            i = i0 + b
            r0 = row_base + i * _K
            nb = 1 - b

            @pl.when(i >= 1)
            def _():
                pltpu.make_async_copy(
                    idx_hbm.at[pl.ds(0, _K)], idx_v.at[b], isem
                ).wait()

            @pl.when(i >= 2)
            def _():
                pltpu.make_async_copy(
                    rows_v.at[b], out_hbm.at[pl.ds(0, blk)], ssems[b]
                ).wait()

            issue_gathers(b)

            @pl.when(i >= 1)
            def _():
                drain_gathers(nb)
                pltpu.async_copy(
                    rows_v.at[nb],
                    out_hbm.at[pl.ds((r0 - _K) * _G, blk)],
                    ssems[nb],
                )

            @pl.when(i + 1 < steps)
            def _():
                pltpu.async_copy(
                    idx_hbm.at[pl.ds(r0 + _K, _K)], idx_v.at[nb], isem
                )

    drain_gathers(1)
    pltpu.async_copy(
        rows_v.at[1],
        out_hbm.at[pl.ds((row_base + (steps - 1) * _K) * _G, blk)],
        ssems[1],
    )
    for b in range(2):
        pltpu.make_async_copy(
            rows_v.at[b], out_hbm.at[pl.ds(0, blk)], ssems[b]
        ).wait()


def kernel(x, table):
    b, l = x.shape
    v, d = table.shape
    n = b * l
    blk = _K * _G
    assert n % (_NUM_WORKERS * blk) == 0
    steps = n // (_NUM_WORKERS * blk)
    assert steps % 2 == 0
    rows_per_worker = n // (_NUM_WORKERS * _G)

    idx2d = x.reshape(n // _G, _G).astype(jnp.int32)

    mesh = plsc.VectorSubcoreMesh(core_axis_name="c", subcore_axis_name="s")
    emb = pl.kernel(
        functools.partial(
            _emb_kernel, steps=steps, rows_per_worker=rows_per_worker
        ),
        out_type=jax.ShapeDtypeStruct((n, d), jnp.float32),
        mesh=mesh,
        scratch_types=[
            pltpu.VMEM((2, _K, _G), jnp.int32),
            pltpu.VMEM((2, blk, d), jnp.float32),
            pltpu.SemaphoreType.DMA,
            pltpu.SemaphoreType.DMA,
            pltpu.SemaphoreType.DMA,
            pltpu.SemaphoreType.DMA,
            pltpu.SemaphoreType.DMA,
        ],
        compiler_params=pltpu.CompilerParams(use_tc_tiling_on_sc=False),
    )
    out = emb(idx2d, table)
    return out.reshape(b, l * d)


# speedup vs baseline: 10.5283x; 1.0007x over previous
import functools

import jax
import jax.numpy as jnp
from jax import lax
from jax.experimental import pallas as pl
from jax.experimental.pallas import tpu as pltpu
from jax.experimental.pallas import tpu_sc as plsc

_NUM_WORKERS = 32
_R = 8


def _emb_kernel(x_hbm, table_hbm, out_hbm, idx_v, rows_v, gsem0, gsem1,
                isem, ssem0, ssem1, *, steps, seq_len, d):
    wid = lax.axis_index("s") * 2 + lax.axis_index("c")
    row_base = wid * (steps * _R)
    blk = _R * seq_len
    gsems = (gsem0, gsem1)
    ssems = (ssem0, ssem1)
    splits = []
    off = 0
    while off < seq_len:
        g = min(128, seq_len - off)
        splits.append((off, g))
        off += g

    def issue_gathers(b, issue):
        for r in range(_R):
            for (o, g) in splits:
                src = table_hbm.at[idx_v.at[b, r, pl.ds(o, g)]]
                dst = rows_v.at[b].at[pl.ds(r * seq_len + o, g)]
                cp = (pltpu.async_copy(src, dst, gsems[b]) if issue else
                      pltpu.make_async_copy(src, dst, gsems[b]).wait())

    pltpu.sync_copy(x_hbm.at[pl.ds(row_base, _R)], idx_v.at[0])

    @pl.loop(0, steps, step=2)
    def pair(i0):
        for b in range(2):
            i = i0 + b
            brow = row_base + i * _R
            nb = 1 - b

            @pl.when(i >= 1)
            def _():
                pltpu.make_async_copy(
                    x_hbm.at[pl.ds(0, _R)], idx_v.at[b], isem
                ).wait()

            @pl.when(i >= 2)
            def _():
                pltpu.make_async_copy(
                    rows_v.at[b], out_hbm.at[pl.ds(0, blk)], ssems[b]
                ).wait()

            issue_gathers(b, True)

            @pl.when(i >= 1)
            def _():
                issue_gathers(nb, False)
                pltpu.async_copy(
                    rows_v.at[nb],
                    out_hbm.at[pl.ds((brow - _R) * seq_len, blk)],
                    ssems[nb],
                )

            @pl.when(i + 1 < steps)
            def _():
                pltpu.async_copy(
                    x_hbm.at[pl.ds(brow + _R, _R)], idx_v.at[nb], isem
                )

    issue_gathers(1, False)
    pltpu.async_copy(
        rows_v.at[1],
        out_hbm.at[pl.ds((row_base + (steps - 1) * _R) * seq_len, blk)],
        ssems[1],
    )
    for b in range(2):
        pltpu.make_async_copy(
            rows_v.at[b], out_hbm.at[pl.ds(0, blk)], ssems[b]
        ).wait()


def kernel(x, table):
    bsz, l = x.shape
    v, d = table.shape
    assert bsz % (_NUM_WORKERS * _R) == 0
    steps = bsz // (_NUM_WORKERS * _R)
    assert steps % 2 == 0
    blk = _R * l

    mesh = plsc.VectorSubcoreMesh(core_axis_name="c", subcore_axis_name="s")
    emb = pl.kernel(
        functools.partial(_emb_kernel, steps=steps, seq_len=l, d=d),
        out_type=jax.ShapeDtypeStruct((bsz * l, d), jnp.float32),
        mesh=mesh,
        scratch_types=[
            pltpu.VMEM((2, _R, l), jnp.int32),
            pltpu.VMEM((2, blk, d), jnp.float32),
            pltpu.SemaphoreType.DMA,
            pltpu.SemaphoreType.DMA,
            pltpu.SemaphoreType.DMA,
            pltpu.SemaphoreType.DMA,
            pltpu.SemaphoreType.DMA,
        ],
        compiler_params=pltpu.CompilerParams(use_tc_tiling_on_sc=False),
    )
    return emb(x.astype(jnp.int32), table).reshape(bsz, l * d)
